# initial kernel scaffold (unmeasured)
import jax
import jax.numpy as jnp
from jax import lax
from jax.experimental import pallas as pl
from jax.experimental.pallas import tpu as pltpu

N_DEV = 4
MC = 1024
KL = 1024
N = 8192
NB = 4
NBS = N // NB


def kernel(x, w_mat, scale_x, scale_w):
    def body(x_ref, w_ref, sx_ref, sw_ref, out_ref,
             comm_ref, stage_ref, accblk_ref,
             send_sems, recv_sems, local_sem, credit_sem):
        my = lax.axis_index("i")
        left = (my + N_DEV - 1) % N_DEV
        right = (my + 1) % N_DEV

        def gemm_block(c, nb):
            xb = x_ref[pl.ds(c * MC, MC), :].astype(jnp.bfloat16)
            wb = w_ref[:, nb * NBS:(nb + 1) * NBS].astype(jnp.bfloat16)
            return jnp.dot(xb, wb, preferred_element_type=jnp.float32)

        def store_chunk(c, slot):
            for nb in range(NB):
                accblk_ref[...] = gemm_block(c, nb)
                cp = pltpu.make_async_copy(
                    accblk_ref,
                    comm_ref.at[slot, :, pl.ds(nb * NBS, NBS)],
                    local_sem,
                )
                cp.start()
                cp.wait()

        def add_chunk(c, slot):
            for nb in range(NB):
                cin = pltpu.make_async_copy(
                    comm_ref.at[slot, :, pl.ds(nb * NBS, NBS)],
                    stage_ref, local_sem,
                )
                cin.start()
                g = gemm_block(c, nb)
                cin.wait()
                accblk_ref[...] = g + stage_ref[...]
                cout = pltpu.make_async_copy(
                    accblk_ref,
                    comm_ref.at[slot, :, pl.ds(nb * NBS, NBS)],
                    local_sem,
                )
                cout.start()
                cout.wait()

        def finalize(slot):
            scale = sx_ref[0] * sw_ref[0]
            for nb in range(NB):
                cin = pltpu.make_async_copy(
                    comm_ref.at[slot, :, pl.ds(nb * NBS, NBS)],
                    stage_ref, local_sem,
                )
                cin.start()
                g = gemm_block(my, nb)
                cin.wait()
                y = (g + stage_ref[...]) * scale
                accblk_ref[...] = y * jax.nn.sigmoid(y)
                cout = pltpu.make_async_copy(
                    accblk_ref,
                    out_ref.at[:, pl.ds(nb * NBS, NBS)],
                    local_sem,
                )
                cout.start()
                cout.wait()

        barrier_sem = pltpu.get_barrier_semaphore()
        for nbr in (left, right):
            pl.semaphore_signal(
                barrier_sem, inc=1,
                device_id=(nbr,), device_id_type=pl.DeviceIdType.MESH,
            )
        pl.semaphore_wait(barrier_sem, 2)

        store_chunk((my + N_DEV - 1) % N_DEV, 0)

        for s in range(N_DEV - 1):
            send_slot = s % 2
            recv_slot = (s + 1) % 2
            if s >= 1:
                pl.semaphore_wait(credit_sem, 1)
            rdma = pltpu.make_async_remote_copy(
                src_ref=comm_ref.at[send_slot],
                dst_ref=comm_ref.at[recv_slot],
                send_sem=send_sems.at[send_slot],
                recv_sem=recv_sems.at[recv_slot],
                device_id=(right,),
                device_id_type=pl.DeviceIdType.MESH,
            )
            rdma.start()
            rdma.wait()
            if s < N_DEV - 2:
                pl.semaphore_signal(
                    credit_sem, inc=1,
                    device_id=(left,), device_id_type=pl.DeviceIdType.MESH,
                )
                add_chunk((my + 2 * N_DEV - 2 - s) % N_DEV, recv_slot)
            else:
                finalize(recv_slot)

    out_shape = jax.ShapeDtypeStruct((MC, N), jnp.float32)
    return pl.pallas_call(
        body,
        out_shape=out_shape,
        in_specs=[
            pl.BlockSpec(memory_space=pltpu.VMEM),
            pl.BlockSpec(memory_space=pltpu.VMEM),
            pl.BlockSpec(memory_space=pltpu.SMEM),
            pl.BlockSpec(memory_space=pltpu.SMEM),
        ],
        out_specs=pl.BlockSpec(memory_space=pltpu.HBM),
        scratch_shapes=[
            pltpu.HBM((2, MC, N), jnp.float32),
            pltpu.VMEM((MC, NBS), jnp.float32),
            pltpu.VMEM((MC, NBS), jnp.float32),
            pltpu.SemaphoreType.DMA((2,)),
            pltpu.SemaphoreType.DMA((2,)),
            pltpu.SemaphoreType.DMA,
            pltpu.SemaphoreType.REGULAR,
        ],
        compiler_params=pltpu.CompilerParams(collective_id=0),
    )(x, w_mat, scale_x, scale_w)


# baseline (device time: 1286837 ns/iter reference)
import jax
import jax.numpy as jnp
from jax import lax
from jax.experimental import pallas as pl
from jax.experimental.pallas import tpu as pltpu

N_DEV = 4
MC = 1024
KL = 1024
N = 8192
NB = 4
NBS = N // NB


def kernel(x, w_mat, scale_x, scale_w):
    def body(x_ref, w_ref, sx_ref, sw_ref, out_ref,
             comm_ref, stage_ref, accblk_ref,
             send_sems, recv_sems, local_sem, credit_sem):
        my = lax.axis_index("i")
        left = (my + N_DEV - 1) % N_DEV
        right = (my + 1) % N_DEV

        def gemm_block(c, nb):
            xb = x_ref[pl.ds(c * MC, MC), :].astype(jnp.bfloat16)
            wb = w_ref[:, nb * NBS:(nb + 1) * NBS].astype(jnp.bfloat16)
            return jnp.dot(xb, wb, preferred_element_type=jnp.float32)

        def store_chunk(c, slot):
            for nb in range(NB):
                accblk_ref[...] = gemm_block(c, nb)
                cp = pltpu.make_async_copy(
                    accblk_ref,
                    comm_ref.at[slot, :, pl.ds(nb * NBS, NBS)],
                    local_sem,
                )
                cp.start()
                cp.wait()

        def add_chunk(c, slot):
            for nb in range(NB):
                cin = pltpu.make_async_copy(
                    comm_ref.at[slot, :, pl.ds(nb * NBS, NBS)],
                    stage_ref, local_sem,
                )
                cin.start()
                g = gemm_block(c, nb)
                cin.wait()
                accblk_ref[...] = g + stage_ref[...]
                cout = pltpu.make_async_copy(
                    accblk_ref,
                    comm_ref.at[slot, :, pl.ds(nb * NBS, NBS)],
                    local_sem,
                )
                cout.start()
                cout.wait()

        def finalize(slot):
            scale = sx_ref[0] * sw_ref[0]
            for nb in range(NB):
                cin = pltpu.make_async_copy(
                    comm_ref.at[slot, :, pl.ds(nb * NBS, NBS)],
                    stage_ref, local_sem,
                )
                cin.start()
                g = gemm_block(my, nb)
                cin.wait()
                y = (g + stage_ref[...]) * scale
                accblk_ref[...] = y * jax.nn.sigmoid(y)
                cout = pltpu.make_async_copy(
                    accblk_ref,
                    out_ref.at[:, pl.ds(nb * NBS, NBS)],
                    local_sem,
                )
                cout.start()
                cout.wait()

        barrier_sem = pltpu.get_barrier_semaphore()
        for nbr in (left, right):
            pl.semaphore_signal(
                barrier_sem, inc=1,
                device_id=(nbr,), device_id_type=pl.DeviceIdType.MESH,
            )
        pl.semaphore_wait(barrier_sem, 2)

        store_chunk((my + N_DEV - 1) % N_DEV, 0)

        for s in range(N_DEV - 1):
            send_slot = s % 2
            recv_slot = (s + 1) % 2
            if s >= 1:
                pl.semaphore_wait(credit_sem, 1)
            rdma = pltpu.make_async_remote_copy(
                src_ref=comm_ref.at[send_slot],
                dst_ref=comm_ref.at[recv_slot],
                send_sem=send_sems.at[send_slot],
                recv_sem=recv_sems.at[recv_slot],
                device_id=(right,),
                device_id_type=pl.DeviceIdType.MESH,
            )
            rdma.start()
            rdma.wait()
            if s < N_DEV - 2:
                pl.semaphore_signal(
                    credit_sem, inc=1,
                    device_id=(left,), device_id_type=pl.DeviceIdType.MESH,
                )
                add_chunk((my + 2 * N_DEV - 2 - s) % N_DEV, recv_slot)
            else:
                finalize(recv_slot)

    out_shape = (
        jax.ShapeDtypeStruct((MC, N), jnp.float32),
        jax.ShapeDtypeStruct((2, MC, N), jnp.float32),
    )
    out, _ = pl.pallas_call(
        body,
        out_shape=out_shape,
        in_specs=[
            pl.BlockSpec(memory_space=pltpu.VMEM),
            pl.BlockSpec(memory_space=pltpu.VMEM),
            pl.BlockSpec(memory_space=pltpu.SMEM),
            pl.BlockSpec(memory_space=pltpu.SMEM),
        ],
        out_specs=(
            pl.BlockSpec(memory_space=pltpu.HBM),
            pl.BlockSpec(memory_space=pltpu.HBM),
        ),
        scratch_shapes=[
            pltpu.VMEM((MC, NBS), jnp.float32),
            pltpu.VMEM((MC, NBS), jnp.float32),
            pltpu.SemaphoreType.DMA((2,)),
            pltpu.SemaphoreType.DMA((2,)),
            pltpu.SemaphoreType.DMA,
            pltpu.SemaphoreType.REGULAR,
        ],
        compiler_params=pltpu.CompilerParams(collective_id=0),
    )(x, w_mat, scale_x, scale_w)
    return out


# device time: 748322 ns/iter; 1.7196x vs baseline; 1.7196x over previous
import jax
import jax.numpy as jnp
from jax import lax
from jax.experimental import pallas as pl
from jax.experimental.pallas import tpu as pltpu

N_DEV = 4
MC = 1024
KL = 1024
N = 8192
NH = N // 2
NBS = 2048
NBH = NH // NBS


def kernel(x, w_mat, scale_x, scale_w):
    def body(x_ref, w_ref, sx_ref, sw_ref,
             out_ref, commr_ref, comml_ref,
             stage_ref, accblk_ref,
             send_r, recv_r, send_l, recv_l,
             local_sem, credit_r, credit_l):
        my = lax.axis_index("i")
        left = (my + N_DEV - 1) % N_DEV
        right = (my + 1) % N_DEV

        def gemm_block(c, col):
            xb = x_ref[pl.ds(c * MC, MC), :].astype(jnp.bfloat16)
            wb = w_ref[:, pl.ds(col, NBS)].astype(jnp.bfloat16)
            return jnp.dot(xb, wb, preferred_element_type=jnp.float32)

        def store_chunk(c, comm, slot, base):
            for nb in range(NBH):
                accblk_ref[...] = gemm_block(c, base + nb * NBS)
                cp = pltpu.make_async_copy(
                    accblk_ref,
                    comm.at[slot, :, pl.ds(nb * NBS, NBS)],
                    local_sem,
                )
                cp.start()
                cp.wait()

        def add_chunk(c, comm, slot, base):
            for nb in range(NBH):
                cin = pltpu.make_async_copy(
                    comm.at[slot, :, pl.ds(nb * NBS, NBS)],
                    stage_ref, local_sem,
                )
                cin.start()
                g = gemm_block(c, base + nb * NBS)
                cin.wait()
                accblk_ref[...] = g + stage_ref[...]
                cout = pltpu.make_async_copy(
                    accblk_ref,
                    comm.at[slot, :, pl.ds(nb * NBS, NBS)],
                    local_sem,
                )
                cout.start()
                cout.wait()

        def finalize(comm, slot, base):
            scale = sx_ref[0] * sw_ref[0]
            for nb in range(NBH):
                cin = pltpu.make_async_copy(
                    comm.at[slot, :, pl.ds(nb * NBS, NBS)],
                    stage_ref, local_sem,
                )
                cin.start()
                g = gemm_block(my, base + nb * NBS)
                cin.wait()
                y = (g + stage_ref[...]) * scale
                accblk_ref[...] = y * jax.nn.sigmoid(y)
                cout = pltpu.make_async_copy(
                    accblk_ref,
                    out_ref.at[:, pl.ds(base + nb * NBS, NBS)],
                    local_sem,
                )
                cout.start()
                cout.wait()

        barrier_sem = pltpu.get_barrier_semaphore()
        for nbr in (left, right):
            pl.semaphore_signal(
                barrier_sem, inc=1,
                device_id=(nbr,), device_id_type=pl.DeviceIdType.MESH,
            )
        pl.semaphore_wait(barrier_sem, 2)

        store_chunk((my + N_DEV - 1) % N_DEV, commr_ref, 0, 0)
        store_chunk((my + 1) % N_DEV, comml_ref, 0, NH)

        for s in range(N_DEV - 1):
            ss = s % 2
            rs = (s + 1) % 2
            if s >= 1:
                pl.semaphore_wait(credit_r, 1)
                pl.semaphore_wait(credit_l, 1)
            rdma_r = pltpu.make_async_remote_copy(
                src_ref=commr_ref.at[ss],
                dst_ref=commr_ref.at[rs],
                send_sem=send_r.at[ss],
                recv_sem=recv_r.at[rs],
                device_id=(right,),
                device_id_type=pl.DeviceIdType.MESH,
            )
            rdma_l = pltpu.make_async_remote_copy(
                src_ref=comml_ref.at[ss],
                dst_ref=comml_ref.at[rs],
                send_sem=send_l.at[ss],
                recv_sem=recv_l.at[rs],
                device_id=(left,),
                device_id_type=pl.DeviceIdType.MESH,
            )
            rdma_r.start()
            rdma_l.start()
            rdma_r.wait()
            rdma_l.wait()
            if s < N_DEV - 2:
                pl.semaphore_signal(
                    credit_r, inc=1,
                    device_id=(left,), device_id_type=pl.DeviceIdType.MESH,
                )
                pl.semaphore_signal(
                    credit_l, inc=1,
                    device_id=(right,), device_id_type=pl.DeviceIdType.MESH,
                )
                add_chunk((my + 2 * N_DEV - 2 - s) % N_DEV, commr_ref, rs, 0)
                add_chunk((my + 2 + s) % N_DEV, comml_ref, rs, NH)
            else:
                finalize(commr_ref, rs, 0)
                finalize(comml_ref, rs, NH)

    out_shape = (
        jax.ShapeDtypeStruct((MC, N), jnp.float32),
        jax.ShapeDtypeStruct((2, MC, NH), jnp.float32),
        jax.ShapeDtypeStruct((2, MC, NH), jnp.float32),
    )
    out, _, _ = pl.pallas_call(
        body,
        out_shape=out_shape,
        in_specs=[
            pl.BlockSpec(memory_space=pltpu.VMEM),
            pl.BlockSpec(memory_space=pltpu.VMEM),
            pl.BlockSpec(memory_space=pltpu.SMEM),
            pl.BlockSpec(memory_space=pltpu.SMEM),
        ],
        out_specs=(
            pl.BlockSpec(memory_space=pltpu.HBM),
            pl.BlockSpec(memory_space=pltpu.HBM),
            pl.BlockSpec(memory_space=pltpu.HBM),
        ),
        scratch_shapes=[
            pltpu.VMEM((MC, NBS), jnp.float32),
            pltpu.VMEM((MC, NBS), jnp.float32),
            pltpu.SemaphoreType.DMA((2,)),
            pltpu.SemaphoreType.DMA((2,)),
            pltpu.SemaphoreType.DMA((2,)),
            pltpu.SemaphoreType.DMA((2,)),
            pltpu.SemaphoreType.DMA,
            pltpu.SemaphoreType.REGULAR,
            pltpu.SemaphoreType.REGULAR,
        ],
        compiler_params=pltpu.CompilerParams(collective_id=0),
    )(x, w_mat, scale_x, scale_w)
    return out


# device time: 605088 ns/iter; 2.1267x vs baseline; 1.2367x over previous
import jax
import jax.numpy as jnp
from jax import lax
from jax.experimental import pallas as pl
from jax.experimental.pallas import tpu as pltpu

N_DEV = 4
MC = 1024
KL = 1024
N = 8192
NH = N // 2
NBS = 2048
P = NH // NBS


def kernel(x, w_mat, scale_x, scale_w):
    def body(x_ref, w_ref, sx_ref, sw_ref,
             out_ref, commr_ref, comml_ref,
             stage_ref, accblk_ref,
             send_r, recv_r, send_l, recv_l,
             local_sem, credit_r, credit_l):
        my = lax.axis_index("i")
        left = (my + N_DEV - 1) % N_DEV
        right = (my + 1) % N_DEV

        rings = (
            (commr_ref, send_r, recv_r, credit_r, right, left, 0,
             (my + N_DEV - 1) % N_DEV,
             lambda s: (my + 2 * N_DEV - 2 - s) % N_DEV),
            (comml_ref, send_l, recv_l, credit_l, left, right, NH,
             (my + 1) % N_DEV,
             lambda s: (my + 2 + s) % N_DEV),
        )

        def gemm_block(c, col):
            xb = x_ref[pl.ds(c * MC, MC), :].astype(jnp.bfloat16)
            wb = w_ref[:, pl.ds(col, NBS)].astype(jnp.bfloat16)
            return jnp.dot(xb, wb, preferred_element_type=jnp.float32)

        def piece_rdma(ring, s, p):
            comm, sends, recvs = ring[0], ring[1], ring[2]
            ss, rs = s % 2, (s + 1) % 2
            return pltpu.make_async_remote_copy(
                src_ref=comm.at[ss, :, pl.ds(p * NBS, NBS)],
                dst_ref=comm.at[rs, :, pl.ds(p * NBS, NBS)],
                send_sem=sends.at[ss, p],
                recv_sem=recvs.at[rs, p],
                device_id=(ring[4],),
                device_id_type=pl.DeviceIdType.MESH,
            )

        def stage_in(comm, slot, p):
            cp = pltpu.make_async_copy(
                comm.at[slot, :, pl.ds(p * NBS, NBS)], stage_ref, local_sem)
            cp.start()
            return cp

        def flush_accblk(dst, col):
            cp = pltpu.make_async_copy(
                accblk_ref, dst.at[:, pl.ds(col, NBS)], local_sem)
            cp.start()
            cp.wait()

        barrier_sem = pltpu.get_barrier_semaphore()
        for nbr in (left, right):
            pl.semaphore_signal(
                barrier_sem, inc=1,
                device_id=(nbr,), device_id_type=pl.DeviceIdType.MESH,
            )
        pl.semaphore_wait(barrier_sem, 2)

        for p in range(P):
            for ring in rings:
                comm, base, c0 = ring[0], ring[6], ring[7]
                accblk_ref[...] = gemm_block(c0, base + p * NBS)
                flush_accblk(comm.at[0], p * NBS)
                piece_rdma(ring, 0, p).start()

        scale = sx_ref[0] * sw_ref[0]
        for s in range(N_DEV - 1):
            rs = (s + 1) % 2
            for p in range(P):
                for ring in rings:
                    comm, credit, ups, base, chunk_at = (
                        ring[0], ring[3], ring[5], ring[6], ring[8])
                    d = piece_rdma(ring, s, p)
                    d.wait_recv()
                    d.wait_send()
                    if s < N_DEV - 2:
                        pl.semaphore_signal(
                            credit, inc=1,
                            device_id=(ups,),
                            device_id_type=pl.DeviceIdType.MESH,
                        )
                        cin = stage_in(comm, rs, p)
                        g = gemm_block(chunk_at(s), base + p * NBS)
                        cin.wait()
                        accblk_ref[...] = g + stage_ref[...]
                        flush_accblk(comm.at[rs], p * NBS)
                        pl.semaphore_wait(credit, 1)
                        piece_rdma(ring, s + 1, p).start()
                    else:
                        cin = stage_in(comm, rs, p)
                        g = gemm_block(my, base + p * NBS)
                        cin.wait()
                        y = (g + stage_ref[...]) * scale
                        accblk_ref[...] = y * jax.nn.sigmoid(y)
                        flush_accblk(out_ref, base + p * NBS)

    out_shape = (
        jax.ShapeDtypeStruct((MC, N), jnp.float32),
        jax.ShapeDtypeStruct((2, MC, NH), jnp.float32),
        jax.ShapeDtypeStruct((2, MC, NH), jnp.float32),
    )
    out, _, _ = pl.pallas_call(
        body,
        out_shape=out_shape,
        in_specs=[
            pl.BlockSpec(memory_space=pltpu.VMEM),
            pl.BlockSpec(memory_space=pltpu.VMEM),
            pl.BlockSpec(memory_space=pltpu.SMEM),
            pl.BlockSpec(memory_space=pltpu.SMEM),
        ],
        out_specs=(
            pl.BlockSpec(memory_space=pltpu.HBM),
            pl.BlockSpec(memory_space=pltpu.HBM),
            pl.BlockSpec(memory_space=pltpu.HBM),
        ),
        scratch_shapes=[
            pltpu.VMEM((MC, NBS), jnp.float32),
            pltpu.VMEM((MC, NBS), jnp.float32),
            pltpu.SemaphoreType.DMA((2, P)),
            pltpu.SemaphoreType.DMA((2, P)),
            pltpu.SemaphoreType.DMA((2, P)),
            pltpu.SemaphoreType.DMA((2, P)),
            pltpu.SemaphoreType.DMA,
            pltpu.SemaphoreType.REGULAR,
            pltpu.SemaphoreType.REGULAR,
        ],
        compiler_params=pltpu.CompilerParams(collective_id=0),
    )(x, w_mat, scale_x, scale_w)
    return out


# device time: 604840 ns/iter; 2.1276x vs baseline; 1.0004x over previous
import jax
import jax.numpy as jnp
from jax import lax
from jax.experimental import pallas as pl
from jax.experimental.pallas import tpu as pltpu

N_DEV = 4
MC = 1024
KL = 1024
N = 8192
NH = N // 2
NBS = 2048
P = NH // NBS


def kernel(x, w_mat, scale_x, scale_w):
    def body(x_ref, w_ref, sx_ref, sw_ref,
             out_ref, commr_ref, comml_ref,
             stage_ref, accblk_ref,
             send_r, recv_r, send_l, recv_l,
             local_sem, credit_r, credit_l):
        my = lax.axis_index("i")
        left = (my + N_DEV - 1) % N_DEV
        right = (my + 1) % N_DEV

        rings = (
            (commr_ref, send_r, recv_r, credit_r, right, left, 0,
             (my + N_DEV - 1) % N_DEV,
             lambda s: (my + 2 * N_DEV - 2 - s) % N_DEV),
            (comml_ref, send_l, recv_l, credit_l, left, right, NH,
             (my + 1) % N_DEV,
             lambda s: (my + 2 + s) % N_DEV),
        )

        def gemm_block(c, col):
            xb = x_ref[pl.ds(c * MC, MC), :].astype(jnp.bfloat16)
            wb = w_ref[:, pl.ds(col, NBS)].astype(jnp.bfloat16)
            return jnp.dot(xb, wb, preferred_element_type=jnp.float32)

        def piece_rdma(ring, s, p):
            comm, sends, recvs = ring[0], ring[1], ring[2]
            ss, rs = s % 2, (s + 1) % 2
            return pltpu.make_async_remote_copy(
                src_ref=comm.at[ss, p],
                dst_ref=comm.at[rs, p],
                send_sem=sends.at[ss, p],
                recv_sem=recvs.at[rs, p],
                device_id=(ring[4],),
                device_id_type=pl.DeviceIdType.MESH,
            )

        def stage_in(comm, slot, p):
            cp = pltpu.make_async_copy(comm.at[slot, p], stage_ref, local_sem)
            cp.start()
            return cp

        def flush_accblk(dst):
            cp = pltpu.make_async_copy(accblk_ref, dst, local_sem)
            cp.start()
            cp.wait()

        barrier_sem = pltpu.get_barrier_semaphore()
        for nbr in (left, right):
            pl.semaphore_signal(
                barrier_sem, inc=1,
                device_id=(nbr,), device_id_type=pl.DeviceIdType.MESH,
            )
        pl.semaphore_wait(barrier_sem, 2)

        for p in range(P):
            for ring in rings:
                comm, base, c0 = ring[0], ring[6], ring[7]
                accblk_ref[...] = gemm_block(c0, base + p * NBS)
                flush_accblk(comm.at[0, p])
                piece_rdma(ring, 0, p).start()

        scale = sx_ref[0] * sw_ref[0]
        for s in range(N_DEV - 1):
            rs = (s + 1) % 2
            for p in range(P):
                for ring in rings:
                    comm, credit, ups, base, chunk_at = (
                        ring[0], ring[3], ring[5], ring[6], ring[8])
                    d = piece_rdma(ring, s, p)
                    d.wait_recv()
                    d.wait_send()
                    if s < N_DEV - 2:
                        pl.semaphore_signal(
                            credit, inc=1,
                            device_id=(ups,),
                            device_id_type=pl.DeviceIdType.MESH,
                        )
                        cin = stage_in(comm, rs, p)
                        g = gemm_block(chunk_at(s), base + p * NBS)
                        cin.wait()
                        accblk_ref[...] = g + stage_ref[...]
                        flush_accblk(comm.at[rs, p])
                        pl.semaphore_wait(credit, 1)
                        piece_rdma(ring, s + 1, p).start()
                    else:
                        cin = stage_in(comm, rs, p)
                        g = gemm_block(my, base + p * NBS)
                        cin.wait()
                        y = (g + stage_ref[...]) * scale
                        accblk_ref[...] = y * jax.nn.sigmoid(y)
                        flush_accblk(
                            out_ref.at[:, pl.ds(base + p * NBS, NBS)])

    out_shape = (
        jax.ShapeDtypeStruct((MC, N), jnp.float32),
        jax.ShapeDtypeStruct((2, P, MC, NBS), jnp.float32),
        jax.ShapeDtypeStruct((2, P, MC, NBS), jnp.float32),
    )
    out, _, _ = pl.pallas_call(
        body,
        out_shape=out_shape,
        in_specs=[
            pl.BlockSpec(memory_space=pltpu.VMEM),
            pl.BlockSpec(memory_space=pltpu.VMEM),
            pl.BlockSpec(memory_space=pltpu.SMEM),
            pl.BlockSpec(memory_space=pltpu.SMEM),
        ],
        out_specs=(
            pl.BlockSpec(memory_space=pltpu.HBM),
            pl.BlockSpec(memory_space=pltpu.HBM),
            pl.BlockSpec(memory_space=pltpu.HBM),
        ),
        scratch_shapes=[
            pltpu.VMEM((MC, NBS), jnp.float32),
            pltpu.VMEM((MC, NBS), jnp.float32),
            pltpu.SemaphoreType.DMA((2, P)),
            pltpu.SemaphoreType.DMA((2, P)),
            pltpu.SemaphoreType.DMA((2, P)),
            pltpu.SemaphoreType.DMA((2, P)),
            pltpu.SemaphoreType.DMA,
            pltpu.SemaphoreType.REGULAR,
            pltpu.SemaphoreType.REGULAR,
        ],
        compiler_params=pltpu.CompilerParams(collective_id=0),
    )(x, w_mat, scale_x, scale_w)
    return out


# device time: 331385 ns/iter; 3.8832x vs baseline; 1.8252x over previous
import jax
import jax.numpy as jnp
from jax import lax
from jax.experimental import pallas as pl
from jax.experimental.pallas import tpu as pltpu

N_DEV = 4
MC = 1024
KL = 1024
N = 8192
NH = N // 2
NBS = 2048
P = NH // NBS


def kernel(x, w_mat, scale_x, scale_w):
    def body(x_ref, w_ref, sx_ref, sw_ref,
             out_ref, commr_ref, comml_ref,
             stage_ref, accblk_ref, outblk_ref,
             send_r, recv_r, send_l, recv_l,
             local_sem, credit_r, credit_l):
        my = lax.axis_index("i")
        left = (my + N_DEV - 1) % N_DEV
        right = (my + 1) % N_DEV

        rings = (
            (commr_ref, send_r, recv_r, credit_r, right, left, 0,
             (my + N_DEV - 1) % N_DEV,
             lambda s: (my + 2 * N_DEV - 2 - s) % N_DEV),
            (comml_ref, send_l, recv_l, credit_l, left, right, NH,
             (my + 1) % N_DEV,
             lambda s: (my + 2 + s) % N_DEV),
        )

        def gemm_block(c, col):
            xb = x_ref[pl.ds(c * MC, MC), :].astype(jnp.bfloat16)
            wb = w_ref[:, pl.ds(col, NBS)].astype(jnp.bfloat16)
            return jnp.dot(xb, wb, preferred_element_type=jnp.float32)

        def piece_rdma(ring, s, p):
            comm, sends, recvs = ring[0], ring[1], ring[2]
            ss, rs = s % 2, (s + 1) % 2
            return pltpu.make_async_remote_copy(
                src_ref=comm.at[ss, p],
                dst_ref=comm.at[rs, p],
                send_sem=sends.at[ss, p],
                recv_sem=recvs.at[rs, p],
                device_id=(ring[4],),
                device_id_type=pl.DeviceIdType.MESH,
            )

        def stage_in(comm, slot, p):
            cp = pltpu.make_async_copy(comm.at[slot, p], stage_ref, local_sem)
            cp.start()
            return cp

        def flush_accblk(dst):
            cp = pltpu.make_async_copy(accblk_ref, dst, local_sem)
            cp.start()
            cp.wait()

        barrier_sem = pltpu.get_barrier_semaphore()
        for nbr in (left, right):
            pl.semaphore_signal(
                barrier_sem, inc=1,
                device_id=(nbr,), device_id_type=pl.DeviceIdType.MESH,
            )
        pl.semaphore_wait(barrier_sem, 2)

        for p in range(P):
            for ring in rings:
                comm, base, c0 = ring[0], ring[6], ring[7]
                accblk_ref[...] = gemm_block(c0, base + p * NBS).astype(
                    jnp.bfloat16)
                flush_accblk(comm.at[0, p])
                piece_rdma(ring, 0, p).start()

        scale = sx_ref[0] * sw_ref[0]
        for s in range(N_DEV - 1):
            rs = (s + 1) % 2
            for p in range(P):
                for ring in rings:
                    comm, credit, ups, base, chunk_at = (
                        ring[0], ring[3], ring[5], ring[6], ring[8])
                    d = piece_rdma(ring, s, p)
                    d.wait_recv()
                    d.wait_send()
                    if s < N_DEV - 2:
                        pl.semaphore_signal(
                            credit, inc=1,
                            device_id=(ups,),
                            device_id_type=pl.DeviceIdType.MESH,
                        )
                        cin = stage_in(comm, rs, p)
                        g = gemm_block(chunk_at(s), base + p * NBS)
                        cin.wait()
                        accblk_ref[...] = (
                            g + stage_ref[...].astype(jnp.float32)
                        ).astype(jnp.bfloat16)
                        flush_accblk(comm.at[rs, p])
                        pl.semaphore_wait(credit, 1)
                        piece_rdma(ring, s + 1, p).start()
                    else:
                        cin = stage_in(comm, rs, p)
                        g = gemm_block(my, base + p * NBS)
                        cin.wait()
                        y = (g + stage_ref[...].astype(jnp.float32)) * scale
                        outblk_ref[...] = y * jax.nn.sigmoid(y)
                        cp = pltpu.make_async_copy(
                            outblk_ref,
                            out_ref.at[:, pl.ds(base + p * NBS, NBS)],
                            local_sem)
                        cp.start()
                        cp.wait()

    out_shape = (
        jax.ShapeDtypeStruct((MC, N), jnp.float32),
        jax.ShapeDtypeStruct((2, P, MC, NBS), jnp.bfloat16),
        jax.ShapeDtypeStruct((2, P, MC, NBS), jnp.bfloat16),
    )
    out, _, _ = pl.pallas_call(
        body,
        out_shape=out_shape,
        in_specs=[
            pl.BlockSpec(memory_space=pltpu.VMEM),
            pl.BlockSpec(memory_space=pltpu.VMEM),
            pl.BlockSpec(memory_space=pltpu.SMEM),
            pl.BlockSpec(memory_space=pltpu.SMEM),
        ],
        out_specs=(
            pl.BlockSpec(memory_space=pltpu.HBM),
            pl.BlockSpec(memory_space=pltpu.HBM),
            pl.BlockSpec(memory_space=pltpu.HBM),
        ),
        scratch_shapes=[
            pltpu.VMEM((MC, NBS), jnp.bfloat16),
            pltpu.VMEM((MC, NBS), jnp.bfloat16),
            pltpu.VMEM((MC, NBS), jnp.float32),
            pltpu.SemaphoreType.DMA((2, P)),
            pltpu.SemaphoreType.DMA((2, P)),
            pltpu.SemaphoreType.DMA((2, P)),
            pltpu.SemaphoreType.DMA((2, P)),
            pltpu.SemaphoreType.DMA,
            pltpu.SemaphoreType.REGULAR,
            pltpu.SemaphoreType.REGULAR,
        ],
        compiler_params=pltpu.CompilerParams(collective_id=0),
    )(x, w_mat, scale_x, scale_w)
    return out


# device time: 320622 ns/iter; 4.0136x vs baseline; 1.0336x over previous
import jax
import jax.numpy as jnp
from jax import lax
from jax.experimental import pallas as pl
from jax.experimental.pallas import tpu as pltpu

N_DEV = 4
MC = 1024
KL = 1024
N = 8192
NH = N // 2
NBS = 1024
P = NH // NBS


def kernel(x, w_mat, scale_x, scale_w):
    def body(x_ref, w_ref, sx_ref, sw_ref,
             out_ref, commr_ref, comml_ref,
             stage_ref, accblk_ref, outblk_ref,
             send_r, recv_r, send_l, recv_l,
             local_sem, credit_r, credit_l):
        my = lax.axis_index("i")
        left = (my + N_DEV - 1) % N_DEV
        right = (my + 1) % N_DEV

        rings = (
            (commr_ref, send_r, recv_r, credit_r, right, left, 0,
             (my + N_DEV - 1) % N_DEV,
             lambda s: (my + 2 * N_DEV - 2 - s) % N_DEV),
            (comml_ref, send_l, recv_l, credit_l, left, right, NH,
             (my + 1) % N_DEV,
             lambda s: (my + 2 + s) % N_DEV),
        )

        def gemm_block(c, col):
            xb = x_ref[pl.ds(c * MC, MC), :].astype(jnp.bfloat16)
            wb = w_ref[:, pl.ds(col, NBS)].astype(jnp.bfloat16)
            return jnp.dot(xb, wb, preferred_element_type=jnp.float32)

        def piece_rdma(ring, s, p):
            comm, sends, recvs = ring[0], ring[1], ring[2]
            ss, rs = s % 2, (s + 1) % 2
            return pltpu.make_async_remote_copy(
                src_ref=comm.at[ss, p],
                dst_ref=comm.at[rs, p],
                send_sem=sends.at[ss, p],
                recv_sem=recvs.at[rs, p],
                device_id=(ring[4],),
                device_id_type=pl.DeviceIdType.MESH,
            )

        def stage_in(comm, slot, p):
            cp = pltpu.make_async_copy(comm.at[slot, p], stage_ref, local_sem)
            cp.start()
            return cp

        def flush_accblk(dst):
            cp = pltpu.make_async_copy(accblk_ref, dst, local_sem)
            cp.start()
            cp.wait()

        barrier_sem = pltpu.get_barrier_semaphore()
        for nbr in (left, right):
            pl.semaphore_signal(
                barrier_sem, inc=1,
                device_id=(nbr,), device_id_type=pl.DeviceIdType.MESH,
            )
        pl.semaphore_wait(barrier_sem, 2)

        for p in range(P):
            for ring in rings:
                comm, base, c0 = ring[0], ring[6], ring[7]
                accblk_ref[...] = gemm_block(c0, base + p * NBS).astype(
                    jnp.bfloat16)
                flush_accblk(comm.at[0, p])
                piece_rdma(ring, 0, p).start()

        scale = sx_ref[0] * sw_ref[0]
        for s in range(N_DEV - 1):
            rs = (s + 1) % 2
            for p in range(P):
                for ring in rings:
                    comm, credit, ups, base, chunk_at = (
                        ring[0], ring[3], ring[5], ring[6], ring[8])
                    d = piece_rdma(ring, s, p)
                    d.wait_recv()
                    d.wait_send()
                    if s < N_DEV - 2:
                        pl.semaphore_signal(
                            credit, inc=1,
                            device_id=(ups,),
                            device_id_type=pl.DeviceIdType.MESH,
                        )
                        cin = stage_in(comm, rs, p)
                        g = gemm_block(chunk_at(s), base + p * NBS)
                        cin.wait()
                        accblk_ref[...] = (
                            g + stage_ref[...].astype(jnp.float32)
                        ).astype(jnp.bfloat16)
                        flush_accblk(comm.at[rs, p])
                        pl.semaphore_wait(credit, 1)
                        piece_rdma(ring, s + 1, p).start()
                    else:
                        cin = stage_in(comm, rs, p)
                        g = gemm_block(my, base + p * NBS)
                        cin.wait()
                        y = (g + stage_ref[...].astype(jnp.float32)) * scale
                        outblk_ref[...] = y * jax.nn.sigmoid(y)
                        cp = pltpu.make_async_copy(
                            outblk_ref,
                            out_ref.at[:, pl.ds(base + p * NBS, NBS)],
                            local_sem)
                        cp.start()
                        cp.wait()

    out_shape = (
        jax.ShapeDtypeStruct((MC, N), jnp.float32),
        jax.ShapeDtypeStruct((2, P, MC, NBS), jnp.bfloat16),
        jax.ShapeDtypeStruct((2, P, MC, NBS), jnp.bfloat16),
    )
    out, _, _ = pl.pallas_call(
        body,
        out_shape=out_shape,
        in_specs=[
            pl.BlockSpec(memory_space=pltpu.VMEM),
            pl.BlockSpec(memory_space=pltpu.VMEM),
            pl.BlockSpec(memory_space=pltpu.SMEM),
            pl.BlockSpec(memory_space=pltpu.SMEM),
        ],
        out_specs=(
            pl.BlockSpec(memory_space=pltpu.HBM),
            pl.BlockSpec(memory_space=pltpu.HBM),
            pl.BlockSpec(memory_space=pltpu.HBM),
        ),
        scratch_shapes=[
            pltpu.VMEM((MC, NBS), jnp.bfloat16),
            pltpu.VMEM((MC, NBS), jnp.bfloat16),
            pltpu.VMEM((MC, NBS), jnp.float32),
            pltpu.SemaphoreType.DMA((2, P)),
            pltpu.SemaphoreType.DMA((2, P)),
            pltpu.SemaphoreType.DMA((2, P)),
            pltpu.SemaphoreType.DMA((2, P)),
            pltpu.SemaphoreType.DMA,
            pltpu.SemaphoreType.REGULAR,
            pltpu.SemaphoreType.REGULAR,
        ],
        compiler_params=pltpu.CompilerParams(collective_id=0),
    )(x, w_mat, scale_x, scale_w)
    return out


# device time: 314811 ns/iter; 4.0876x vs baseline; 1.0185x over previous
import jax
import jax.numpy as jnp
from jax import lax
from jax.experimental import pallas as pl
from jax.experimental.pallas import tpu as pltpu

N_DEV = 4
MC = 1024
KL = 1024
N = 8192
NH = N // 2
NBS = 512
P = NH // NBS


def kernel(x, w_mat, scale_x, scale_w):
    def body(x_ref, w_ref, sx_ref, sw_ref,
             out_ref, commr_ref, comml_ref,
             stage_ref, accblk_ref, outblk_ref,
             send_r, recv_r, send_l, recv_l,
             local_sem, credit_r, credit_l):
        my = lax.axis_index("i")
        left = (my + N_DEV - 1) % N_DEV
        right = (my + 1) % N_DEV

        rings = (
            (commr_ref, send_r, recv_r, credit_r, right, left, 0,
             (my + N_DEV - 1) % N_DEV,
             lambda s: (my + 2 * N_DEV - 2 - s) % N_DEV),
            (comml_ref, send_l, recv_l, credit_l, left, right, NH,
             (my + 1) % N_DEV,
             lambda s: (my + 2 + s) % N_DEV),
        )

        def gemm_block(c, col):
            xb = x_ref[pl.ds(c * MC, MC), :].astype(jnp.bfloat16)
            wb = w_ref[:, pl.ds(col, NBS)].astype(jnp.bfloat16)
            return jnp.dot(xb, wb, preferred_element_type=jnp.float32)

        def piece_rdma(ring, s, p):
            comm, sends, recvs = ring[0], ring[1], ring[2]
            ss, rs = s % 2, (s + 1) % 2
            return pltpu.make_async_remote_copy(
                src_ref=comm.at[ss, p],
                dst_ref=comm.at[rs, p],
                send_sem=sends.at[ss, p],
                recv_sem=recvs.at[rs, p],
                device_id=(ring[4],),
                device_id_type=pl.DeviceIdType.MESH,
            )

        def stage_in(comm, slot, p):
            cp = pltpu.make_async_copy(comm.at[slot, p], stage_ref, local_sem)
            cp.start()
            return cp

        def flush_accblk(dst):
            cp = pltpu.make_async_copy(accblk_ref, dst, local_sem)
            cp.start()
            cp.wait()

        barrier_sem = pltpu.get_barrier_semaphore()
        for nbr in (left, right):
            pl.semaphore_signal(
                barrier_sem, inc=1,
                device_id=(nbr,), device_id_type=pl.DeviceIdType.MESH,
            )
        pl.semaphore_wait(barrier_sem, 2)

        for p in range(P):
            for ring in rings:
                comm, base, c0 = ring[0], ring[6], ring[7]
                accblk_ref[...] = gemm_block(c0, base + p * NBS).astype(
                    jnp.bfloat16)
                flush_accblk(comm.at[0, p])
                piece_rdma(ring, 0, p).start()

        scale = sx_ref[0] * sw_ref[0]
        for s in range(N_DEV - 1):
            rs = (s + 1) % 2
            for p in range(P):
                for ring in rings:
                    comm, credit, ups, base, chunk_at = (
                        ring[0], ring[3], ring[5], ring[6], ring[8])
                    d = piece_rdma(ring, s, p)
                    d.wait_recv()
                    d.wait_send()
                    if s < N_DEV - 2:
                        pl.semaphore_signal(
                            credit, inc=1,
                            device_id=(ups,),
                            device_id_type=pl.DeviceIdType.MESH,
                        )
                        cin = stage_in(comm, rs, p)
                        g = gemm_block(chunk_at(s), base + p * NBS)
                        cin.wait()
                        accblk_ref[...] = (
                            g + stage_ref[...].astype(jnp.float32)
                        ).astype(jnp.bfloat16)
                        flush_accblk(comm.at[rs, p])
                        pl.semaphore_wait(credit, 1)
                        piece_rdma(ring, s + 1, p).start()
                    else:
                        cin = stage_in(comm, rs, p)
                        g = gemm_block(my, base + p * NBS)
                        cin.wait()
                        y = (g + stage_ref[...].astype(jnp.float32)) * scale
                        outblk_ref[...] = y * jax.nn.sigmoid(y)
                        cp = pltpu.make_async_copy(
                            outblk_ref,
                            out_ref.at[:, pl.ds(base + p * NBS, NBS)],
                            local_sem)
                        cp.start()
                        cp.wait()

    out_shape = (
        jax.ShapeDtypeStruct((MC, N), jnp.float32),
        jax.ShapeDtypeStruct((2, P, MC, NBS), jnp.bfloat16),
        jax.ShapeDtypeStruct((2, P, MC, NBS), jnp.bfloat16),
    )
    out, _, _ = pl.pallas_call(
        body,
        out_shape=out_shape,
        in_specs=[
            pl.BlockSpec(memory_space=pltpu.VMEM),
            pl.BlockSpec(memory_space=pltpu.VMEM),
            pl.BlockSpec(memory_space=pltpu.SMEM),
            pl.BlockSpec(memory_space=pltpu.SMEM),
        ],
        out_specs=(
            pl.BlockSpec(memory_space=pltpu.HBM),
            pl.BlockSpec(memory_space=pltpu.HBM),
            pl.BlockSpec(memory_space=pltpu.HBM),
        ),
        scratch_shapes=[
            pltpu.VMEM((MC, NBS), jnp.bfloat16),
            pltpu.VMEM((MC, NBS), jnp.bfloat16),
            pltpu.VMEM((MC, NBS), jnp.float32),
            pltpu.SemaphoreType.DMA((2, P)),
            pltpu.SemaphoreType.DMA((2, P)),
            pltpu.SemaphoreType.DMA((2, P)),
            pltpu.SemaphoreType.DMA((2, P)),
            pltpu.SemaphoreType.DMA,
            pltpu.SemaphoreType.REGULAR,
            pltpu.SemaphoreType.REGULAR,
        ],
        compiler_params=pltpu.CompilerParams(collective_id=0),
    )(x, w_mat, scale_x, scale_w)
    return out
